# Initial kernel scaffold; baseline (speedup 1.0000x reference)
import jax
import jax.numpy as jnp
from jax.experimental import pallas as pl


def kernel(x_customer, x_article, edge_index_c2a, edge_index_a2c, edge_label_index, Wl_ca1, bl_ca1, Wr_ca1, Wl_ac1, bl_ac1, Wr_ac1, Wl_ca2, bl_ca2, Wr_ca2, Wl_ac2, bl_ac2, Wr_ac2, Wl_ca3, bl_ca3, Wr_ca3, Wl_ac3, bl_ac3, Wr_ac3, Wd1, bd1, Wd2, bd2, Wd3, bd3):
    raise NotImplementedError("write your pallas kernel here")


# speedup vs baseline: 1.4798x; 1.4798x over previous
import functools

import jax
import jax.numpy as jnp
from jax import lax
from jax.experimental import pallas as pl
from jax.experimental.pallas import tpu as pltpu
from jax.experimental.pallas import tpu_sc as plsc

NC, NS, LANES = 2, 16, 16
CH = 128

f32 = jnp.float32


def _rup(x, m):
    return (x + m - 1) // m * m


def _mesh():
    return plsc.VectorSubcoreMesh(
        core_axis_name="c", subcore_axis_name="s", num_cores=NC, num_subcores=NS
    )


def _row_chunks(rows):
    out = []
    off = 0
    while off < rows:
        sz = min(CH, rows - off)
        out.append((off, sz))
        off += sz
    return out


@functools.lru_cache(maxsize=None)
def _make_counts(na_pad, nc_pad, e_pad):
    ept = e_pad // NS
    n_chunks = ept // CH

    @functools.partial(
        pl.kernel,
        out_type=(
            jax.ShapeDtypeStruct((na_pad, CH), f32),
            jax.ShapeDtypeStruct((nc_pad, CH), f32),
        ),
        mesh=_mesh(),
        scratch_types=dict(
            idx_v=pltpu.VMEM((CH,), jnp.int32),
            ones_v=pltpu.VMEM((CH, CH), f32),
            cnt_sh=pltpu.VMEM_SHARED((na_pad, CH), f32),
        ),
    )
    def counts_kernel(dst_a, dst_c, out_a, out_c, idx_v, ones_v, cnt_sh):
        cid = lax.axis_index("c")
        sid = lax.axis_index("s")

        def fill(val):
            def body(i, _):
                for j in range(CH // LANES):
                    ones_v[i, pl.ds(j * LANES, LANES)] = jnp.full((LANES,), val, f32)
                return 0

            lax.fori_loop(0, CH, body, 0)

        def run(dst_ref, out_ref, n_pad):
            rows_pt = n_pad // NS
            pieces = _row_chunks(rows_pt)

            fill(0.0)
            for off, sz in pieces:
                pltpu.sync_copy(
                    ones_v.at[pl.ds(0, sz)], cnt_sh.at[pl.ds(sid * rows_pt + off, sz)]
                )
            plsc.subcore_barrier()
            fill(1.0)

            def chunk_body(i, _):
                base = sid * ept + i * CH
                pltpu.sync_copy(dst_ref.at[pl.ds(base, CH)], idx_v)
                pltpu.sync_copy(ones_v, cnt_sh.at[idx_v], add=True)
                return 0

            lax.fori_loop(0, n_chunks, chunk_body, 0)
            plsc.subcore_barrier()

            for off, sz in pieces:
                r0 = sid * rows_pt + off
                pltpu.sync_copy(cnt_sh.at[pl.ds(r0, sz)], out_ref.at[pl.ds(r0, sz)])

        @pl.when(cid == 0)
        def _():
            run(dst_a, out_a, na_pad)

        @pl.when(cid == 1)
        def _():
            run(dst_c, out_c, nc_pad)

    return counts_kernel


@functools.lru_cache(maxsize=None)
def _make_agg(n_dst_pad, n_units, e_pad):
    ept = e_pad // NS
    n_chunks = ept // CH
    rows_pt = n_dst_pad // NS
    pieces = _row_chunks(rows_pt)
    G = NC * n_units

    scratch = dict(
        gidx_v=pltpu.VMEM((CH,), jnp.int32),
        dst_v=pltpu.VMEM((CH,), jnp.int32),
        rows_v=pltpu.VMEM((CH, CH), f32),
    )
    for u in range(n_units):
        scratch["agg_sh%d" % u] = pltpu.VMEM_SHARED((n_dst_pad, CH), f32)

    @functools.partial(
        pl.kernel,
        out_type=jax.ShapeDtypeStruct((G, n_dst_pad, CH), f32),
        mesh=_mesh(),
        scratch_types=scratch,
    )
    def agg_kernel(xg, gsrc, dst, out, gidx_v, dst_v, rows_v, **agg_kw):
        aggs = [agg_kw["agg_sh%d" % u] for u in range(n_units)]
        cid = lax.axis_index("c")
        sid = lax.axis_index("s")

        def zfill(i, _):
            for j in range(CH // LANES):
                rows_v[i, pl.ds(j * LANES, LANES)] = jnp.zeros((LANES,), f32)
            return 0

        lax.fori_loop(0, CH, zfill, 0)
        for u in range(n_units):
            for off, sz in pieces:
                pltpu.sync_copy(
                    rows_v.at[pl.ds(0, sz)], aggs[u].at[pl.ds(sid * rows_pt + off, sz)]
                )
        plsc.subcore_barrier()

        def chunk_body(i, _):
            base = sid * ept + i * CH
            pltpu.sync_copy(dst.at[pl.ds(base, CH)], dst_v)
            for u in range(n_units):
                pltpu.sync_copy(gsrc.at[cid * n_units + u, pl.ds(base, CH)], gidx_v)
                pltpu.sync_copy(xg.at[gidx_v], rows_v)
                pltpu.sync_copy(rows_v, aggs[u].at[dst_v], add=True)
            return 0

        lax.fori_loop(0, n_chunks, chunk_body, 0)
        plsc.subcore_barrier()

        for u in range(n_units):
            for off, sz in pieces:
                r0 = sid * rows_pt + off
                pltpu.sync_copy(
                    aggs[u].at[pl.ds(r0, sz)], out.at[cid * n_units + u, pl.ds(r0, sz)]
                )

    return agg_kernel


@functools.lru_cache(maxsize=None)
def _make_dec_gather(b_pad, d):
    rows_pt = b_pad // NS
    n_chunks = rows_pt // CH

    @functools.partial(
        pl.kernel,
        out_type=(
            jax.ShapeDtypeStruct((b_pad, d), f32),
            jax.ShapeDtypeStruct((b_pad, d), f32),
        ),
        mesh=_mesh(),
        scratch_types=dict(
            idx_v=pltpu.VMEM((CH,), jnp.int32),
            rows_v=pltpu.VMEM((CH, d), f32),
        ),
    )
    def gather_kernel(t1, t2, idx1, idx2, out1, out2, idx_v, rows_v):
        cid = lax.axis_index("c")
        sid = lax.axis_index("s")

        def run(t_ref, i_ref, o_ref):
            def chunk_body(i, _):
                base = sid * rows_pt + i * CH
                pltpu.sync_copy(i_ref.at[pl.ds(base, CH)], idx_v)
                pltpu.sync_copy(t_ref.at[idx_v], rows_v)
                pltpu.sync_copy(rows_v, o_ref.at[pl.ds(base, CH)])
                return 0

            lax.fori_loop(0, n_chunks, chunk_body, 0)

        @pl.when(cid == 0)
        def _():
            run(t1, idx1, out1)

        @pl.when(cid == 1)
        def _():
            run(t2, idx2, out2)

    return gather_kernel


@functools.lru_cache(maxsize=None)
def _make_mm(n, n_pad, ngroups, din, dout, relu, bm):
    def body(agg_ref, cnt_ref, x_ref, w_ref, b_ref, o_ref):
        inv = 1.0 / jnp.maximum(cnt_ref[:, :1], 1.0)
        acc = jnp.dot(x_ref[...], w_ref[ngroups * CH :], preferred_element_type=f32)
        for g in range(ngroups):
            acc += jnp.dot(
                agg_ref[g] * inv, w_ref[g * CH : (g + 1) * CH], preferred_element_type=f32
            )
        acc += b_ref[...]
        if relu:
            acc = jnp.maximum(acc, 0.0)
        o_ref[...] = acc

    return pl.pallas_call(
        body,
        grid=(n // bm,),
        in_specs=[
            pl.BlockSpec((ngroups, bm, CH), lambda i: (0, i, 0)),
            pl.BlockSpec((bm, CH), lambda i: (i, 0)),
            pl.BlockSpec((bm, din), lambda i: (i, 0)),
            pl.BlockSpec((ngroups * CH + din, dout), lambda i: (0, 0)),
            pl.BlockSpec((1, dout), lambda i: (0, 0)),
        ],
        out_specs=pl.BlockSpec((bm, dout), lambda i: (i, 0)),
        out_shape=jax.ShapeDtypeStruct((n, dout), f32),
    )


@functools.lru_cache(maxsize=None)
def _make_dec_mm(b_pad, d, h1, h2, bm):
    def body(g1_ref, g2_ref, w1a_ref, w1b_ref, b1_ref, w2_ref, b2_ref, w3_ref, b3_ref, o_ref):
        z = jnp.dot(g1_ref[...], w1a_ref[...], preferred_element_type=f32)
        z += jnp.dot(g2_ref[...], w1b_ref[...], preferred_element_type=f32)
        z = jnp.maximum(z + b1_ref[...], 0.0)
        z = jnp.maximum(jnp.dot(z, w2_ref[...], preferred_element_type=f32) + b2_ref[...], 0.0)
        o_ref[...] = jnp.dot(z, w3_ref[...], preferred_element_type=f32) + b3_ref[...]

    full = lambda shape: pl.BlockSpec(shape, lambda i: tuple(0 for _ in shape))
    return pl.pallas_call(
        body,
        grid=(b_pad // bm,),
        in_specs=[
            pl.BlockSpec((bm, d), lambda i: (i, 0)),
            pl.BlockSpec((bm, d), lambda i: (i, 0)),
            full((d, h1)),
            full((d, h1)),
            full((1, h1)),
            full((h1, h2)),
            full((1, h2)),
            full((h2, 128)),
            full((1, 128)),
        ],
        out_specs=pl.BlockSpec((bm, 128), lambda i: (i, 0)),
        out_shape=jax.ShapeDtypeStruct((b_pad, 128), f32),
    )


def _pad1(a, n, val):
    return jnp.concatenate([a, jnp.full((n - a.shape[0],), val, a.dtype)])


def _padw(w, rows, cols):
    return jnp.pad(w, ((0, rows - w.shape[0]), (0, cols - w.shape[1])))


def kernel(x_customer, x_article, edge_index_c2a, edge_index_a2c, edge_label_index,
           Wl_ca1, bl_ca1, Wr_ca1, Wl_ac1, bl_ac1, Wr_ac1,
           Wl_ca2, bl_ca2, Wr_ca2, Wl_ac2, bl_ac2, Wr_ac2,
           Wl_ca3, bl_ca3, Wr_ca3, Wl_ac3, bl_ac3, Wr_ac3,
           Wd1, bd1, Wd2, bd2, Wd3, bd3):
    nc, d0 = x_customer.shape
    na = x_article.shape[0]
    e = edge_index_c2a.shape[1]
    b = edge_label_index.shape[1]

    tile_m = NS * CH
    e_pad = _rup(e, tile_m)
    b_pad = _rup(b, tile_m)
    na_pad = _rup(na + 1, NS * 8)
    nc_pad = _rup(nc + 1, NS * 8)

    src_a = _pad1(edge_index_c2a[0], e_pad, 0)
    dst_a = _pad1(edge_index_c2a[1], e_pad, na)
    src_c = _pad1(edge_index_a2c[0], e_pad, 0)
    dst_c = _pad1(edge_index_a2c[1], e_pad, nc)
    gsrc_a = {g: jnp.stack([g * src_a + j for j in range(g)]) for g in (2, 4)}
    gsrc_c = {g: jnp.stack([g * src_c + j for j in range(g)]) for g in (2, 4)}

    cnt_a, cnt_c = _make_counts(na_pad, nc_pad, e_pad)(dst_a, dst_c)

    dp = 512
    wca = [
        jnp.concatenate([Wl_ca1, Wr_ca1], axis=0),
        jnp.concatenate([_padw(Wl_ca2, 512, dp), _padw(Wr_ca2, 512, dp)], axis=0),
        jnp.concatenate([_padw(Wl_ca3, dp, 512), _padw(Wr_ca3, dp, 512)], axis=0),
    ]
    wac = [
        jnp.concatenate([Wl_ac1, Wr_ac1], axis=0),
        jnp.concatenate([_padw(Wl_ac2, 512, dp), _padw(Wr_ac2, 512, dp)], axis=0),
        jnp.concatenate([_padw(Wl_ac3, dp, 512), _padw(Wr_ac3, dp, 512)], axis=0),
    ]
    bca = [bl_ca1.reshape(1, -1), _pad1(bl_ca2, dp, 0.0).reshape(1, -1), bl_ca3.reshape(1, -1)]
    bac = [bl_ac1.reshape(1, -1), _pad1(bl_ac2, dp, 0.0).reshape(1, -1), bl_ac3.reshape(1, -1)]

    zc, za = x_customer, x_article
    bm = 1000
    for l in range(3):
        d = zc.shape[1]
        g = d // CH
        dout = wca[l].shape[1]
        relu = l < 2
        agg_a = _make_agg(na_pad, g // NC, e_pad)(zc.reshape(-1, CH), gsrc_a[g], dst_a)
        agg_c = _make_agg(nc_pad, g // NC, e_pad)(za.reshape(-1, CH), gsrc_c[g], dst_c)
        za_new = _make_mm(na, na_pad, g, d, dout, relu, bm)(agg_a, cnt_a, za, wca[l], bca[l])
        zc_new = _make_mm(nc, nc_pad, g, d, dout, relu, bm)(agg_c, cnt_c, zc, wac[l], bac[l])
        zc, za = zc_new, za_new

    ci = _pad1(edge_label_index[0], b_pad, 0)
    ai = _pad1(edge_label_index[1], b_pad, 0)
    g1, g2 = _make_dec_gather(b_pad, zc.shape[1])(zc, za, ci, ai)
    h1 = Wd1.shape[1]
    h2 = Wd2.shape[1]
    out = _make_dec_mm(b_pad, zc.shape[1], h1, h2, 1024)(
        g1, g2,
        Wd1[: zc.shape[1]], Wd1[zc.shape[1] :],
        bd1.reshape(1, -1),
        Wd2, bd2.reshape(1, -1),
        _padw(Wd3, h2, 128), _pad1(bd3, 128, 0.0).reshape(1, -1),
    )
    return out[:b, 0]


# speedup vs baseline: 1.8221x; 1.2313x over previous
import functools

import jax
import jax.numpy as jnp
from jax import lax
from jax.experimental import pallas as pl
from jax.experimental.pallas import tpu as pltpu
from jax.experimental.pallas import tpu_sc as plsc

NC, NS, LANES = 2, 16, 16
CH = 128

f32 = jnp.float32


def _rup(x, m):
    return (x + m - 1) // m * m


def _mesh():
    return plsc.VectorSubcoreMesh(
        core_axis_name="c", subcore_axis_name="s", num_cores=NC, num_subcores=NS
    )


def _row_chunks(rows):
    out = []
    off = 0
    while off < rows:
        sz = min(CH, rows - off)
        out.append((off, sz))
        off += sz
    return out


@functools.lru_cache(maxsize=None)
def _make_counts(na_pad, nc_pad, e_pad):
    ept = e_pad // NS
    n_chunks = ept // CH

    @functools.partial(
        pl.kernel,
        out_type=(
            jax.ShapeDtypeStruct((na_pad, CH), f32),
            jax.ShapeDtypeStruct((nc_pad, CH), f32),
        ),
        mesh=_mesh(),
        scratch_types=dict(
            idx_v=pltpu.VMEM((e_pad // NS // CH, CH), jnp.int32),
            ones_v=pltpu.VMEM((CH, CH), f32),
            cnt_sh=pltpu.VMEM_SHARED((na_pad, CH), f32),
            sem=pltpu.SemaphoreType.DMA,
        ),
    )
    def counts_kernel(dst_a, dst_c, out_a, out_c, idx_v, ones_v, cnt_sh, sem):
        cid = lax.axis_index("c")
        sid = lax.axis_index("s")

        def fill(val):
            def body(i, _):
                for j in range(CH // LANES):
                    ones_v[i, pl.ds(j * LANES, LANES)] = jnp.full((LANES,), val, f32)
                return 0

            lax.fori_loop(0, CH, body, 0)

        def run(dst_ref, out_ref, n_pad):
            rows_pt = n_pad // NS
            pieces = _row_chunks(rows_pt)

            pltpu.sync_copy(dst_ref.at[sid], idx_v)
            fill(0.0)
            for off, sz in pieces:
                pltpu.sync_copy(
                    ones_v.at[pl.ds(0, sz)], cnt_sh.at[pl.ds(sid * rows_pt + off, sz)]
                )
            plsc.subcore_barrier()
            fill(1.0)

            def fire(i, _):
                pltpu.async_copy(ones_v, cnt_sh.at[idx_v.at[i]], sem, add=True)
                return 0

            lax.fori_loop(0, n_chunks, fire, 0)

            def drain(i, _):
                pltpu.make_async_copy(ones_v, cnt_sh.at[idx_v.at[i]], sem).wait()
                return 0

            lax.fori_loop(0, n_chunks, drain, 0)
            plsc.subcore_barrier()

            for off, sz in pieces:
                r0 = sid * rows_pt + off
                pltpu.sync_copy(cnt_sh.at[pl.ds(r0, sz)], out_ref.at[pl.ds(r0, sz)])

        @pl.when(cid == 0)
        def _():
            run(dst_a, out_a, na_pad)

        @pl.when(cid == 1)
        def _():
            run(dst_c, out_c, nc_pad)

    return counts_kernel


@functools.lru_cache(maxsize=None)
def _make_agg(n_dst_pad, n_units, e_pad):
    ept = e_pad // NS
    n_chunks = ept // CH
    n_pairs = n_chunks // 2
    rows_pt = n_dst_pad // NS
    pieces = _row_chunks(rows_pt)
    G = NC * n_units

    @functools.partial(
        pl.kernel,
        out_type=jax.ShapeDtypeStruct((G, n_dst_pad, CH), f32),
        mesh=_mesh(),
        scratch_types=dict(
            gidx_v=pltpu.VMEM((n_units, n_chunks, CH), jnp.int32),
            dst_v=pltpu.VMEM((n_chunks, CH), jnp.int32),
            buf0=pltpu.VMEM((CH, CH), f32),
            buf1=pltpu.VMEM((CH, CH), f32),
            agg_sh=pltpu.VMEM_SHARED((n_dst_pad, CH), f32),
            gsem0=pltpu.SemaphoreType.DMA,
            gsem1=pltpu.SemaphoreType.DMA,
            ssem0=pltpu.SemaphoreType.DMA,
            ssem1=pltpu.SemaphoreType.DMA,
        ),
    )
    def agg_kernel(xg, gsrc, dst, out, gidx_v, dst_v, buf0, buf1, agg_sh,
                   gsem0, gsem1, ssem0, ssem1):
        cid = lax.axis_index("c")
        sid = lax.axis_index("s")

        pltpu.sync_copy(dst.at[sid], dst_v)
        for u in range(n_units):
            pltpu.sync_copy(gsrc.at[cid * n_units + u, sid], gidx_v.at[u])

        def zfill(i, _):
            for j in range(CH // LANES):
                buf0[i, pl.ds(j * LANES, LANES)] = jnp.zeros((LANES,), f32)
            return 0

        lax.fori_loop(0, CH, zfill, 0)

        def gissue(u, i, buf, sem):
            return pltpu.async_copy(xg.at[gidx_v.at[u, i]], buf, sem)

        def gwait(u, i, buf, sem):
            pltpu.make_async_copy(xg.at[gidx_v.at[u, i]], buf, sem).wait()

        def sissue(u, i, buf, sem):
            return pltpu.async_copy(buf, agg_sh.at[dst_v.at[i]], sem, add=True)

        def swait(u, i, buf, sem):
            pltpu.make_async_copy(buf, agg_sh.at[dst_v.at[i]], sem).wait()

        for u in range(n_units):
            for off, sz in pieces:
                pltpu.sync_copy(
                    buf0.at[pl.ds(0, sz)], agg_sh.at[pl.ds(sid * rows_pt + off, sz)]
                )
            plsc.subcore_barrier()

            gissue(u, 0, buf0, gsem0)

            def pair(k, first, last):
                i0 = 2 * k
                i1 = 2 * k + 1
                gwait(u, i0, buf0, gsem0)
                if not first:
                    swait(u, i1 - 2, buf1, ssem1)
                gissue(u, i1, buf1, gsem1)
                sissue(u, i0, buf0, ssem0)
                gwait(u, i1, buf1, gsem1)
                swait(u, i0, buf0, ssem0)
                if not last:
                    gissue(u, i1 + 1, buf0, gsem0)
                sissue(u, i1, buf1, ssem1)

            pair(0, True, n_pairs == 1)

            def loop_body(k, _):
                pair(k, False, False)
                return 0

            if n_pairs > 2:
                lax.fori_loop(1, n_pairs - 1, loop_body, 0)
            if n_pairs > 1:
                pair(n_pairs - 1, False, True)
            swait(u, n_chunks - 1, buf1, ssem1)
            plsc.subcore_barrier()

            for off, sz in pieces:
                r0 = sid * rows_pt + off
                pltpu.sync_copy(
                    agg_sh.at[pl.ds(r0, sz)], out.at[cid * n_units + u, pl.ds(r0, sz)]
                )
            if u + 1 < n_units:
                lax.fori_loop(0, CH, zfill, 0)

    return agg_kernel


@functools.lru_cache(maxsize=None)
def _make_dec_gather(b_pad, d, cb):
    rows_pt = b_pad // NS
    n_chunks = rows_pt // cb
    n_pairs = n_chunks // 2

    @functools.partial(
        pl.kernel,
        out_type=(
            jax.ShapeDtypeStruct((b_pad, d), f32),
            jax.ShapeDtypeStruct((b_pad, d), f32),
        ),
        mesh=_mesh(),
        scratch_types=dict(
            idx_v=pltpu.VMEM((n_chunks, cb), jnp.int32),
            buf0=pltpu.VMEM((cb, d), f32),
            buf1=pltpu.VMEM((cb, d), f32),
            gsem0=pltpu.SemaphoreType.DMA,
            gsem1=pltpu.SemaphoreType.DMA,
            wsem0=pltpu.SemaphoreType.DMA,
            wsem1=pltpu.SemaphoreType.DMA,
        ),
    )
    def gather_kernel(t1, t2, idx1, idx2, out1, out2, idx_v, buf0, buf1,
                      gsem0, gsem1, wsem0, wsem1):
        cid = lax.axis_index("c")
        sid = lax.axis_index("s")

        def run(t_ref, i_ref, o_ref):
            pltpu.sync_copy(i_ref.at[sid], idx_v)

            def gissue(i, buf, sem):
                pltpu.async_copy(t_ref.at[idx_v.at[i]], buf, sem)

            def gwait(i, buf, sem):
                pltpu.make_async_copy(t_ref.at[idx_v.at[i]], buf, sem).wait()

            def o_slice(i):
                return o_ref.at[pl.ds(sid * rows_pt + i * cb, cb)]

            def wissue(i, buf, sem):
                pltpu.async_copy(buf, o_slice(i), sem)

            def wwait(i, buf, sem):
                pltpu.make_async_copy(buf, o_slice(i), sem).wait()

            gissue(0, buf0, gsem0)

            def pair(k, first, last):
                i0 = 2 * k
                i1 = 2 * k + 1
                gwait(i0, buf0, gsem0)
                if not first:
                    wwait(i1 - 2, buf1, wsem1)
                gissue(i1, buf1, gsem1)
                wissue(i0, buf0, wsem0)
                gwait(i1, buf1, gsem1)
                wwait(i0, buf0, wsem0)
                if not last:
                    gissue(i1 + 1, buf0, gsem0)
                wissue(i1, buf1, wsem1)

            pair(0, True, n_pairs == 1)

            def loop_body(k, _):
                pair(k, False, False)
                return 0

            if n_pairs > 2:
                lax.fori_loop(1, n_pairs - 1, loop_body, 0)
            if n_pairs > 1:
                pair(n_pairs - 1, False, True)
            wwait(n_chunks - 1, buf1, wsem1)

        @pl.when(cid == 0)
        def _():
            run(t1, idx1, out1)

        @pl.when(cid == 1)
        def _():
            run(t2, idx2, out2)

    return gather_kernel


@functools.lru_cache(maxsize=None)
def _make_mm(n, n_pad, ngroups, din, dout, relu, bm):
    def body(agg_ref, cnt_ref, x_ref, w_ref, b_ref, o_ref):
        inv = 1.0 / jnp.maximum(cnt_ref[:, :1], 1.0)
        acc = jnp.dot(x_ref[...], w_ref[ngroups * CH :], preferred_element_type=f32)
        for g in range(ngroups):
            acc += jnp.dot(
                agg_ref[g] * inv, w_ref[g * CH : (g + 1) * CH], preferred_element_type=f32
            )
        acc += b_ref[...]
        if relu:
            acc = jnp.maximum(acc, 0.0)
        o_ref[...] = acc

    return pl.pallas_call(
        body,
        grid=(n // bm,),
        in_specs=[
            pl.BlockSpec((ngroups, bm, CH), lambda i: (0, i, 0)),
            pl.BlockSpec((bm, CH), lambda i: (i, 0)),
            pl.BlockSpec((bm, din), lambda i: (i, 0)),
            pl.BlockSpec((ngroups * CH + din, dout), lambda i: (0, 0)),
            pl.BlockSpec((1, dout), lambda i: (0, 0)),
        ],
        out_specs=pl.BlockSpec((bm, dout), lambda i: (i, 0)),
        out_shape=jax.ShapeDtypeStruct((n, dout), f32),
    )


@functools.lru_cache(maxsize=None)
def _make_dec_mm(b_pad, d, h1, h2, bm):
    def body(g1_ref, g2_ref, w1a_ref, w1b_ref, b1_ref, w2_ref, b2_ref, w3_ref, b3_ref, o_ref):
        z = jnp.dot(g1_ref[...], w1a_ref[...], preferred_element_type=f32)
        z += jnp.dot(g2_ref[...], w1b_ref[...], preferred_element_type=f32)
        z = jnp.maximum(z + b1_ref[...], 0.0)
        z = jnp.maximum(jnp.dot(z, w2_ref[...], preferred_element_type=f32) + b2_ref[...], 0.0)
        o_ref[...] = jnp.dot(z, w3_ref[...], preferred_element_type=f32) + b3_ref[...]

    full = lambda shape: pl.BlockSpec(shape, lambda i: tuple(0 for _ in shape))
    return pl.pallas_call(
        body,
        grid=(b_pad // bm,),
        in_specs=[
            pl.BlockSpec((bm, d), lambda i: (i, 0)),
            pl.BlockSpec((bm, d), lambda i: (i, 0)),
            full((d, h1)),
            full((d, h1)),
            full((1, h1)),
            full((h1, h2)),
            full((1, h2)),
            full((h2, 128)),
            full((1, 128)),
        ],
        out_specs=pl.BlockSpec((bm, 128), lambda i: (i, 0)),
        out_shape=jax.ShapeDtypeStruct((b_pad, 128), f32),
    )


def _pad1(a, n, val):
    return jnp.concatenate([a, jnp.full((n - a.shape[0],), val, a.dtype)])


def _padw(w, rows, cols):
    return jnp.pad(w, ((0, rows - w.shape[0]), (0, cols - w.shape[1])))


def kernel(x_customer, x_article, edge_index_c2a, edge_index_a2c, edge_label_index,
           Wl_ca1, bl_ca1, Wr_ca1, Wl_ac1, bl_ac1, Wr_ac1,
           Wl_ca2, bl_ca2, Wr_ca2, Wl_ac2, bl_ac2, Wr_ac2,
           Wl_ca3, bl_ca3, Wr_ca3, Wl_ac3, bl_ac3, Wr_ac3,
           Wd1, bd1, Wd2, bd2, Wd3, bd3):
    nc, d0 = x_customer.shape
    na = x_article.shape[0]
    e = edge_index_c2a.shape[1]
    b = edge_label_index.shape[1]

    tile_m = NS * CH
    e_pad = _rup(e, tile_m)
    b_pad = _rup(b, tile_m)
    na_pad = _rup(na + 1, NS * 8)
    nc_pad = _rup(nc + 1, NS * 8)

    src_a = _pad1(edge_index_c2a[0], e_pad, 0)
    dst_a = _pad1(edge_index_c2a[1], e_pad, na).reshape(NS, -1, CH)
    src_c = _pad1(edge_index_a2c[0], e_pad, 0)
    dst_c = _pad1(edge_index_a2c[1], e_pad, nc).reshape(NS, -1, CH)
    gsrc_a = {g: jnp.stack([(g * src_a + j).reshape(NS, -1, CH) for j in range(g)])
              for g in (2, 4)}
    gsrc_c = {g: jnp.stack([(g * src_c + j).reshape(NS, -1, CH) for j in range(g)])
              for g in (2, 4)}

    cnt_a, cnt_c = _make_counts(na_pad, nc_pad, e_pad)(dst_a, dst_c)

    dp = 512
    wca = [
        jnp.concatenate([Wl_ca1, Wr_ca1], axis=0),
        jnp.concatenate([_padw(Wl_ca2, 512, dp), _padw(Wr_ca2, 512, dp)], axis=0),
        jnp.concatenate([_padw(Wl_ca3, dp, 512), _padw(Wr_ca3, dp, 512)], axis=0),
    ]
    wac = [
        jnp.concatenate([Wl_ac1, Wr_ac1], axis=0),
        jnp.concatenate([_padw(Wl_ac2, 512, dp), _padw(Wr_ac2, 512, dp)], axis=0),
        jnp.concatenate([_padw(Wl_ac3, dp, 512), _padw(Wr_ac3, dp, 512)], axis=0),
    ]
    bca = [bl_ca1.reshape(1, -1), _pad1(bl_ca2, dp, 0.0).reshape(1, -1), bl_ca3.reshape(1, -1)]
    bac = [bl_ac1.reshape(1, -1), _pad1(bl_ac2, dp, 0.0).reshape(1, -1), bl_ac3.reshape(1, -1)]

    zc, za = x_customer, x_article
    bm = 1000
    for l in range(3):
        d = zc.shape[1]
        g = d // CH
        dout = wca[l].shape[1]
        relu = l < 2
        agg_a = _make_agg(na_pad, g // NC, e_pad)(zc.reshape(-1, CH), gsrc_a[g], dst_a)
        agg_c = _make_agg(nc_pad, g // NC, e_pad)(za.reshape(-1, CH), gsrc_c[g], dst_c)
        za_new = _make_mm(na, na_pad, g, d, dout, relu, bm)(agg_a, cnt_a, za, wca[l], bca[l])
        zc_new = _make_mm(nc, nc_pad, g, d, dout, relu, bm)(agg_c, cnt_c, zc, wac[l], bac[l])
        zc, za = zc_new, za_new

    cb = 64
    ci = _pad1(edge_label_index[0], b_pad, 0).reshape(NS, -1, cb)
    ai = _pad1(edge_label_index[1], b_pad, 0).reshape(NS, -1, cb)
    g1, g2 = _make_dec_gather(b_pad, zc.shape[1], cb)(zc, za, ci, ai)
    h1 = Wd1.shape[1]
    h2 = Wd2.shape[1]
    out = _make_dec_mm(b_pad, zc.shape[1], h1, h2, 1024)(
        g1, g2,
        Wd1[: zc.shape[1]], Wd1[zc.shape[1] :],
        bd1.reshape(1, -1),
        Wd2, bd2.reshape(1, -1),
        _padw(Wd3, h2, 128), _pad1(bd3, 128, 0.0).reshape(1, -1),
    )
    return out[:b, 0]
